# Initial kernel scaffold; baseline (speedup 1.0000x reference)
import jax
import jax.numpy as jnp
from jax.experimental import pallas as pl


def kernel(one_hot, features, gemme_features, a_res, W1, Ws1, b1, W2, Ws2, b2, W3, Ws3, b3, W4, Ws4, b4):
    raise NotImplementedError("write your pallas kernel here")


# speedup vs baseline: 7.9218x; 7.9218x over previous
import functools

import jax
import jax.numpy as jnp
from jax import lax
from jax.experimental import pallas as pl
from jax.experimental.pallas import tpu as pltpu
from jax.experimental.pallas import tpu_sc as plsc

N = 10000
NPAD = 10240
NC, NS = 2, 16
NW = NC * NS
CH = 128
DSC = 128


def _prep_indices(src, dst):
    e = src.shape[0]
    epw = -(-e // NW)
    nchunk = -(-epw // CH)
    nchunk = -(-nchunk // 8) * 8
    total = NW * nchunk * CH
    pad = total - e
    pad_ar = jnp.arange(pad, dtype=jnp.int32)
    src_p = jnp.concatenate([src, pad_ar % N])
    dst_p = jnp.concatenate([dst, N + pad_ar % (NPAD - N)])
    return (src_p.reshape(NW, nchunk, CH), dst_p.reshape(NW, nchunk, CH),
            nchunk)


@functools.cache
def _make_sc_segment_sum(nchunk):
    mesh = plsc.VectorSubcoreMesh(core_axis_name="c", subcore_axis_name="s",
                                  num_cores=NC, num_subcores=NS)
    rpt = NPAD // NS

    @functools.partial(
        pl.kernel,
        out_type=jax.ShapeDtypeStruct((NC, NPAD, DSC), jnp.float32),
        mesh=mesh,
        scratch_types=[
            pltpu.VMEM((nchunk, CH), jnp.int32),
            pltpu.VMEM((nchunk, CH), jnp.int32),
            pltpu.VMEM((CH, DSC), jnp.float32),
            pltpu.VMEM_SHARED((NPAD, DSC), jnp.float32),
            pltpu.SemaphoreType.DMA,
        ],
    )
    def sc_kernel(y_hbm, src_hbm, dst_hbm, init_hbm, out_hbm,
                  src_v, dst_v, rows_v, acc_s, sem):
        cid = lax.axis_index("c")
        sid = lax.axis_index("s")
        wid = sid * NC + cid
        r0 = sid * rpt
        pltpu.sync_copy(init_hbm.at[cid, pl.ds(r0, rpt)],
                        acc_s.at[pl.ds(r0, rpt)])
        pltpu.sync_copy(src_hbm.at[wid], src_v)
        pltpu.sync_copy(dst_hbm.at[wid], dst_v)
        plsc.subcore_barrier()

        def body(j, carry):
            pltpu.async_copy(y_hbm.at[src_v.at[j]], rows_v, sem).wait()
            pltpu.sync_copy(rows_v, acc_s.at[dst_v.at[j]], add=True)
            return carry

        lax.fori_loop(0, nchunk, body, 0)
        plsc.subcore_barrier()
        pltpu.sync_copy(acc_s.at[pl.ds(r0, rpt)],
                        out_hbm.at[cid, pl.ds(r0, rpt)])

    return sc_kernel


def _dense_stage_outputs(x, w_r, ws_r, b_r, y_r, init_r):
    d = w_r.shape[1]
    y = jnp.dot(x, w_r[...], preferred_element_type=jnp.float32)
    s = jnp.dot(x, ws_r[...], preferred_element_type=jnp.float32) + b_r[...]
    zc = jnp.zeros((N, DSC - d), jnp.float32)
    zr = jnp.zeros((NPAD - N, DSC), jnp.float32)
    y_r[...] = jnp.concatenate([y, zc], axis=1)
    init_r[0, :, :] = jnp.concatenate(
        [jnp.concatenate([s, zc], axis=1), zr], axis=0)
    init_r[1, :, :] = jnp.zeros((NPAD, DSC), jnp.float32)


def _dense_first(one_hot, features, w, ws, b):
    def body(oh_r, ft_r, w_r, ws_r, b_r, y_r, init_r):
        x = jnp.concatenate([oh_r[...], ft_r[...]], axis=1)
        _dense_stage_outputs(x, w_r, ws_r, b_r, y_r, init_r)

    return pl.pallas_call(
        body,
        out_shape=(jax.ShapeDtypeStruct((N, DSC), jnp.float32),
                   jax.ShapeDtypeStruct((2, NPAD, DSC), jnp.float32)),
    )(one_hot, features, w, ws, b.reshape(1, -1))


def _dense_mid(parts, w, ws, b):
    din = w.shape[0]

    def body(p_r, w_r, ws_r, b_r, y_r, init_r):
        x = jnp.maximum(p_r[0] + p_r[1], 0.0)[:N, :din]
        _dense_stage_outputs(x, w_r, ws_r, b_r, y_r, init_r)

    return pl.pallas_call(
        body,
        out_shape=(jax.ShapeDtypeStruct((N, DSC), jnp.float32),
                   jax.ShapeDtypeStruct((2, NPAD, DSC), jnp.float32)),
    )(parts, w, ws, b.reshape(1, -1))


def _mean_final(parts):
    def body(p_r, out_r):
        x = jnp.maximum(p_r[0] + p_r[1], 0.0)[:N, :1]
        out_r[...] = jnp.sum(x, keepdims=True) / N

    out = pl.pallas_call(
        body, out_shape=jax.ShapeDtypeStruct((1, 1), jnp.float32))(parts)
    return out[0, 0]


def kernel(one_hot, features, gemme_features, a_res,
           W1, Ws1, b1, W2, Ws2, b2, W3, Ws3, b3, W4, Ws4, b4):
    del gemme_features
    src2, dst2, nchunk = _prep_indices(a_res[0], a_res[1])
    sc_seg = _make_sc_segment_sum(nchunk)

    parts = None
    for li, (w, ws, b) in enumerate(
            ((W1, Ws1, b1), (W2, Ws2, b2), (W3, Ws3, b3), (W4, Ws4, b4))):
        if li == 0:
            y, init = _dense_first(one_hot, features, w, ws, b)
        else:
            y, init = _dense_mid(parts, w, ws, b)
        parts = sc_seg(y, src2, dst2, init)
    return _mean_final(parts)


# speedup vs baseline: 10.8344x; 1.3677x over previous
import functools

import jax
import jax.numpy as jnp
from jax import lax
from jax.experimental import pallas as pl
from jax.experimental.pallas import tpu as pltpu
from jax.experimental.pallas import tpu_sc as plsc

N = 10000
NPAD = 10240
NC, NS = 2, 16
NW = NC * NS
CH = 80
DSC = 128


def _prep_indices(src, dst):
    e = src.shape[0]
    epw = -(-e // NW)
    nchunk = -(-epw // CH)
    nchunk = -(-nchunk // 8) * 8
    total = NW * nchunk * CH
    pad = total - e
    pad_ar = jnp.arange(pad, dtype=jnp.int32)
    src_p = jnp.concatenate([src, pad_ar % N])
    dst_p = jnp.concatenate([dst, N + pad_ar % (NPAD - N)])
    return (src_p.reshape(NW, nchunk * CH), dst_p.reshape(NW, nchunk, CH),
            nchunk)


@functools.cache
def _make_sc_segment_sum(nchunk):
    mesh = plsc.VectorSubcoreMesh(core_axis_name="c", subcore_axis_name="s",
                                  num_cores=NC, num_subcores=NS)
    rpt = NPAD // NS

    @functools.partial(
        pl.kernel,
        out_type=jax.ShapeDtypeStruct((NC, NPAD, DSC), jnp.float32),
        mesh=mesh,
        scratch_types=[
            pltpu.VMEM((nchunk * CH,), jnp.int32),
            pltpu.VMEM((nchunk, CH), jnp.int32),
            pltpu.VMEM((2, CH, DSC), jnp.float32),
            pltpu.VMEM_SHARED((NPAD, DSC), jnp.float32),
            pltpu.SemaphoreType.DMA,
        ],
    )
    def sc_kernel(y_hbm, src_hbm, dst_hbm, init_hbm, out_hbm,
                  src_v, dst_v, rows_v, acc_s, sem_a):
        cid = lax.axis_index("c")
        sid = lax.axis_index("s")
        wid = sid * NC + cid
        r0 = sid * rpt
        rsub = rpt // 8

        def init_body(k, carry):
            rk = r0 + k * rsub
            pltpu.sync_copy(init_hbm.at[cid, pl.ds(rk, rsub)],
                            acc_s.at[pl.ds(rk, rsub)])
            return carry

        lax.fori_loop(0, 8, init_body, 0)
        pltpu.sync_copy(src_hbm.at[wid], src_v)
        pltpu.sync_copy(dst_hbm.at[wid], dst_v)
        plsc.subcore_barrier()

        def start_gather(j, par):
            pltpu.async_copy(y_hbm.at[src_v.at[pl.ds(j * CH, CH)]],
                             rows_v.at[par], sem_a)

        start_gather(0, 0)

        def body(j, carry):
            par = lax.rem(j, 2)

            @pl.when(j + 1 < nchunk)
            def _():
                start_gather(j + 1, 1 - par)

            pltpu.make_async_copy(
                y_hbm.at[src_v.at[pl.ds(j * CH, CH)]],
                rows_v.at[par], sem_a).wait()
            pltpu.sync_copy(rows_v.at[par], acc_s.at[dst_v.at[j]], add=True)
            return carry

        lax.fori_loop(0, nchunk, body, 0)
        plsc.subcore_barrier()

        def out_body(k, carry):
            rk = r0 + k * rsub
            pltpu.sync_copy(acc_s.at[pl.ds(rk, rsub)],
                            out_hbm.at[cid, pl.ds(rk, rsub)])
            return carry

        lax.fori_loop(0, 8, out_body, 0)

    return sc_kernel


def _dense_stage_outputs(x, w_r, ws_r, b_r, y_r, init_r):
    d = w_r.shape[1]
    y = jnp.dot(x, w_r[...], preferred_element_type=jnp.float32)
    s = jnp.dot(x, ws_r[...], preferred_element_type=jnp.float32) + b_r[...]
    zc = jnp.zeros((N, DSC - d), jnp.float32)
    zr = jnp.zeros((NPAD - N, DSC), jnp.float32)
    y_r[...] = jnp.concatenate([y, zc], axis=1)
    init_r[0, :, :] = jnp.concatenate(
        [jnp.concatenate([s, zc], axis=1), zr], axis=0)
    init_r[1, :, :] = jnp.zeros((NPAD, DSC), jnp.float32)


def _dense_first(one_hot, features, w, ws, b):
    def body(oh_r, ft_r, w_r, ws_r, b_r, y_r, init_r):
        x = jnp.concatenate([oh_r[...], ft_r[...]], axis=1)
        _dense_stage_outputs(x, w_r, ws_r, b_r, y_r, init_r)

    return pl.pallas_call(
        body,
        out_shape=(jax.ShapeDtypeStruct((N, DSC), jnp.float32),
                   jax.ShapeDtypeStruct((2, NPAD, DSC), jnp.float32)),
    )(one_hot, features, w, ws, b.reshape(1, -1))


def _dense_mid(parts, w, ws, b):
    din = w.shape[0]

    def body(p_r, w_r, ws_r, b_r, y_r, init_r):
        x = jnp.maximum(p_r[0] + p_r[1], 0.0)[:N, :din]
        _dense_stage_outputs(x, w_r, ws_r, b_r, y_r, init_r)

    return pl.pallas_call(
        body,
        out_shape=(jax.ShapeDtypeStruct((N, DSC), jnp.float32),
                   jax.ShapeDtypeStruct((2, NPAD, DSC), jnp.float32)),
    )(parts, w, ws, b.reshape(1, -1))


def _mean_final(parts):
    def body(p_r, out_r):
        x = jnp.maximum(p_r[0] + p_r[1], 0.0)[:N, :1]
        out_r[...] = jnp.sum(x, keepdims=True) / N

    out = pl.pallas_call(
        body, out_shape=jax.ShapeDtypeStruct((1, 1), jnp.float32))(parts)
    return out[0, 0]


def kernel(one_hot, features, gemme_features, a_res,
           W1, Ws1, b1, W2, Ws2, b2, W3, Ws3, b3, W4, Ws4, b4):
    del gemme_features
    src2, dst2, nchunk = _prep_indices(a_res[0], a_res[1])
    sc_seg = _make_sc_segment_sum(nchunk)

    parts = None
    for li, (w, ws, b) in enumerate(
            ((W1, Ws1, b1), (W2, Ws2, b2), (W3, Ws3, b3), (W4, Ws4, b4))):
        if li == 0:
            y, init = _dense_first(one_hot, features, w, ws, b)
        else:
            y, init = _dense_mid(parts, w, ws, b)
        parts = sc_seg(y, src2, dst2, init)
    return _mean_final(parts)


# speedup vs baseline: 11.7328x; 1.0829x over previous
import functools

import jax
import jax.numpy as jnp
from jax import lax
from jax.experimental import pallas as pl
from jax.experimental.pallas import tpu as pltpu
from jax.experimental.pallas import tpu_sc as plsc

N = 10000
NPAD = 10240
NC, NS = 2, 16
NW = NC * NS
CH = 80
DSC = 128


def _prep_indices(src, dst):
    e = src.shape[0]
    epw = -(-e // NW)
    nchunk = -(-epw // CH)
    nchunk = -(-nchunk // 8) * 8
    total = NW * nchunk * CH
    pad = total - e
    pad_ar = jnp.arange(pad, dtype=jnp.int32)
    src_p = jnp.concatenate([src, pad_ar % N])
    dst_p = jnp.concatenate([dst, N + pad_ar % (NPAD - N)])
    return (src_p.reshape(NW, nchunk * CH), dst_p.reshape(NW, nchunk, CH),
            nchunk)


@functools.cache
def _make_sc_segment_sum(nchunk):
    mesh = plsc.VectorSubcoreMesh(core_axis_name="c", subcore_axis_name="s",
                                  num_cores=NC, num_subcores=NS)
    rpt = NPAD // NS

    @functools.partial(
        pl.kernel,
        out_type=jax.ShapeDtypeStruct((NC, NPAD, DSC), jnp.float32),
        mesh=mesh,
        scratch_types=[
            pltpu.VMEM((nchunk * CH,), jnp.int32),
            pltpu.VMEM((nchunk, CH), jnp.int32),
            pltpu.VMEM((2, CH, DSC), jnp.float32),
            pltpu.VMEM_SHARED((NPAD, DSC), jnp.float32),
            pltpu.SemaphoreType.DMA,
        ],
    )
    def sc_kernel(y_hbm, src_hbm, dst_hbm, init_hbm, out_hbm,
                  src_v, dst_v, rows_v, acc_s, sem_a):
        cid = lax.axis_index("c")
        sid = lax.axis_index("s")
        wid = sid * NC + cid
        r0 = sid * rpt
        rsub = rpt // 8

        def init_body(k, carry):
            rk = r0 + k * rsub
            pltpu.sync_copy(init_hbm.at[cid, pl.ds(rk, rsub)],
                            acc_s.at[pl.ds(rk, rsub)])
            return carry

        lax.fori_loop(0, 8, init_body, 0)
        pltpu.sync_copy(src_hbm.at[wid], src_v)
        pltpu.sync_copy(dst_hbm.at[wid], dst_v)
        plsc.subcore_barrier()

        def start_gather(j, par):
            pltpu.async_copy(y_hbm.at[src_v.at[pl.ds(j * CH, CH)]],
                             rows_v.at[par], sem_a)

        start_gather(0, 0)

        def body(j, carry):
            par = lax.rem(j, 2)

            @pl.when(j + 1 < nchunk)
            def _():
                start_gather(j + 1, 1 - par)

            pltpu.make_async_copy(
                y_hbm.at[src_v.at[pl.ds(j * CH, CH)]],
                rows_v.at[par], sem_a).wait()
            pltpu.sync_copy(rows_v.at[par], acc_s.at[dst_v.at[j]], add=True)
            return carry

        lax.fori_loop(0, nchunk, body, 0)
        plsc.subcore_barrier()

        def out_body(k, carry):
            rk = r0 + k * rsub
            pltpu.sync_copy(acc_s.at[pl.ds(rk, rsub)],
                            out_hbm.at[cid, pl.ds(rk, rsub)])
            return carry

        lax.fori_loop(0, 8, out_body, 0)

    return sc_kernel


@functools.cache
def _make_sc_segment_sum_1d(nchunk):
    mesh = plsc.VectorSubcoreMesh(core_axis_name="c", subcore_axis_name="s",
                                  num_cores=NC, num_subcores=NS)
    rpt = NPAD // NS

    @functools.partial(
        pl.kernel,
        out_type=jax.ShapeDtypeStruct((NC, NPAD), jnp.float32),
        mesh=mesh,
        scratch_types=[
            pltpu.VMEM((nchunk * CH,), jnp.int32),
            pltpu.VMEM((nchunk, CH), jnp.int32),
            pltpu.VMEM((2, CH), jnp.float32),
            pltpu.VMEM_SHARED((NPAD,), jnp.float32),
            pltpu.SemaphoreType.DMA,
        ],
    )
    def sc_kernel(y_hbm, src_hbm, dst_hbm, init_hbm, out_hbm,
                  src_v, dst_v, vals_v, acc_s, sem_a):
        cid = lax.axis_index("c")
        sid = lax.axis_index("s")
        wid = sid * NC + cid
        r0 = sid * rpt
        pltpu.sync_copy(init_hbm.at[cid, pl.ds(r0, rpt)],
                        acc_s.at[pl.ds(r0, rpt)])
        pltpu.sync_copy(src_hbm.at[wid], src_v)
        pltpu.sync_copy(dst_hbm.at[wid], dst_v)
        plsc.subcore_barrier()

        def start_gather(j, par):
            pltpu.async_copy(y_hbm.at[src_v.at[pl.ds(j * CH, CH)]],
                             vals_v.at[par], sem_a)

        start_gather(0, 0)

        def body(j, carry):
            par = lax.rem(j, 2)

            @pl.when(j + 1 < nchunk)
            def _():
                start_gather(j + 1, 1 - par)

            pltpu.make_async_copy(
                y_hbm.at[src_v.at[pl.ds(j * CH, CH)]],
                vals_v.at[par], sem_a).wait()
            pltpu.sync_copy(vals_v.at[par], acc_s.at[dst_v.at[j]], add=True)
            return carry

        lax.fori_loop(0, nchunk, body, 0)
        plsc.subcore_barrier()
        pltpu.sync_copy(acc_s.at[pl.ds(r0, rpt)],
                        out_hbm.at[cid, pl.ds(r0, rpt)])

    return sc_kernel


def _dense_stage_outputs(x, w_r, ws_r, b_r, y_r, init_r):
    d = w_r.shape[1]
    y = jnp.dot(x, w_r[...], preferred_element_type=jnp.float32)
    s = jnp.dot(x, ws_r[...], preferred_element_type=jnp.float32) + b_r[...]
    zc = jnp.zeros((N, DSC - d), jnp.float32)
    zr = jnp.zeros((NPAD - N, DSC), jnp.float32)
    y_r[...] = jnp.concatenate([y, zc], axis=1)
    init_r[0, :, :] = jnp.concatenate(
        [jnp.concatenate([s, zc], axis=1), zr], axis=0)
    init_r[1, :, :] = jnp.zeros((NPAD, DSC), jnp.float32)


def _dense_first(one_hot, features, w, ws, b):
    def body(oh_r, ft_r, w_r, ws_r, b_r, y_r, init_r):
        x = jnp.concatenate([oh_r[...], ft_r[...]], axis=1)
        _dense_stage_outputs(x, w_r, ws_r, b_r, y_r, init_r)

    return pl.pallas_call(
        body,
        out_shape=(jax.ShapeDtypeStruct((N, DSC), jnp.float32),
                   jax.ShapeDtypeStruct((2, NPAD, DSC), jnp.float32)),
    )(one_hot, features, w, ws, b.reshape(1, -1))


def _dense_mid(parts, w, ws, b):
    din = w.shape[0]

    def body(p_r, w_r, ws_r, b_r, y_r, init_r):
        x = jnp.maximum(p_r[0] + p_r[1], 0.0)[:N, :din]
        _dense_stage_outputs(x, w_r, ws_r, b_r, y_r, init_r)

    return pl.pallas_call(
        body,
        out_shape=(jax.ShapeDtypeStruct((N, DSC), jnp.float32),
                   jax.ShapeDtypeStruct((2, NPAD, DSC), jnp.float32)),
    )(parts, w, ws, b.reshape(1, -1))


def _dense_last(parts, w, ws, b):
    din = w.shape[0]

    def body(p_r, w_r, ws_r, b_r, y_r, init_r):
        x = jnp.maximum(p_r[0] + p_r[1], 0.0)[:N, :din]
        y = jnp.dot(x, w_r[...], preferred_element_type=jnp.float32)
        s = jnp.dot(x, ws_r[...], preferred_element_type=jnp.float32) + b_r[...]
        y_r[...] = y
        init_r[0, :, :] = jnp.concatenate(
            [s, jnp.zeros((NPAD - N, 1), jnp.float32)], axis=0)
        init_r[1, :, :] = jnp.zeros((NPAD, 1), jnp.float32)

    y, init = pl.pallas_call(
        body,
        out_shape=(jax.ShapeDtypeStruct((N, 1), jnp.float32),
                   jax.ShapeDtypeStruct((2, NPAD, 1), jnp.float32)),
    )(parts, w, ws, b.reshape(1, -1))
    return y.reshape(N), init.reshape(2, NPAD)


def _mean_final(parts):
    def body(p_r, out_r):
        x = jnp.maximum(p_r[0:1, :] + p_r[1:2, :], 0.0)[:, :N]
        out_r[...] = jnp.sum(x, axis=1, keepdims=True) / N

    out = pl.pallas_call(
        body, out_shape=jax.ShapeDtypeStruct((1, 1), jnp.float32))(parts)
    return out[0, 0]


def kernel(one_hot, features, gemme_features, a_res,
           W1, Ws1, b1, W2, Ws2, b2, W3, Ws3, b3, W4, Ws4, b4):
    del gemme_features
    src2, dst2, nchunk = _prep_indices(a_res[0], a_res[1])
    sc_seg = _make_sc_segment_sum(nchunk)

    parts = None
    for li, (w, ws, b) in enumerate(
            ((W1, Ws1, b1), (W2, Ws2, b2), (W3, Ws3, b3))):
        if li == 0:
            y, init = _dense_first(one_hot, features, w, ws, b)
        else:
            y, init = _dense_mid(parts, w, ws, b)
        parts = sc_seg(y, src2, dst2, init)
    y1d, init1d = _dense_last(parts, W4, Ws4, b4)
    parts1d = _make_sc_segment_sum_1d(nchunk)(y1d, src2, dst2, init1d)
    return _mean_final(parts1d)
